# baseline (speedup 1.0000x reference)
import functools

import jax
import jax.numpy as jnp
from jax import lax
from jax.experimental import pallas as pl
from jax.experimental.pallas import tpu as pltpu
from jax.experimental.pallas import tpu_sc as plsc

N = 10000
E = 320000
D = 128

NC = 2
NS = 16
NW = NC * NS
EPT = E // NW
CH = 80
NCH = EPT // CH
NP = 10240
RPT = NP // NS
ZR = 128

NH = 10240
HPT = NH // NS

_mesh = plsc.VectorSubcoreMesh(core_axis_name="c", subcore_axis_name="s")


@functools.partial(
    pl.kernel,
    out_type=jax.ShapeDtypeStruct((2 * NC * NH,), jnp.float32),
    mesh=_mesh,
    scratch_types=[
        pltpu.VMEM((NCH, CH), jnp.int32),
        pltpu.VMEM((NCH, CH), jnp.int32),
        pltpu.VMEM((CH,), jnp.float32),
        pltpu.VMEM((HPT,), jnp.float32),
        pltpu.VMEM_SHARED((NH,), jnp.float32),
        pltpu.VMEM_SHARED((NH,), jnp.float32),
    ],
)
def _deg_kernel(src_hbm, dst_hbm, out_hbm, src_v, dst_v, ones_v, zeros_v,
                hsrc_sh, hdst_sh):
    c = lax.axis_index("c")
    s = lax.axis_index("s")
    wid = s * NC + c

    pltpu.sync_copy(src_hbm.at[wid], src_v)
    pltpu.sync_copy(dst_hbm.at[wid], dst_v)

    for i in range(CH // 16):
        ones_v[pl.ds(i * 16, 16)] = jnp.ones((16,), jnp.float32)

    def _zero(i, _):
        zeros_v[pl.ds(i * 16, 16)] = jnp.zeros((16,), jnp.float32)
        return 0
    lax.fori_loop(0, HPT // 16, _zero, 0)

    hoff = pl.multiple_of(s * HPT, 128)
    pltpu.sync_copy(zeros_v, hsrc_sh.at[pl.ds(hoff, HPT)])
    pltpu.sync_copy(zeros_v, hdst_sh.at[pl.ds(hoff, HPT)])
    plsc.subcore_barrier()

    def _accum(i, _):
        pltpu.sync_copy(ones_v, hsrc_sh.at[src_v.at[i]], add=True)
        pltpu.sync_copy(ones_v, hdst_sh.at[dst_v.at[i]], add=True)
        return 0
    lax.fori_loop(0, NCH, _accum, 0)

    plsc.subcore_barrier()
    osrc = pl.multiple_of(c * (2 * NH) + s * HPT, 128)
    odst = pl.multiple_of(c * (2 * NH) + NH + s * HPT, 128)
    pltpu.sync_copy(hsrc_sh.at[pl.ds(hoff, HPT)], out_hbm.at[pl.ds(osrc, HPT)])
    pltpu.sync_copy(hdst_sh.at[pl.ds(hoff, HPT)], out_hbm.at[pl.ds(odst, HPT)])


HALF = NP // NC
AR = 5248
ART = AR // NS
WBT = HALF // NS
CH2 = 128
EPAD = 20480
EP = EPAD * NS
NCH2 = EPAD // CH2
NBUF = 2


@functools.partial(
    pl.kernel,
    out_type=jax.ShapeDtypeStruct((NP, D), jnp.float32),
    mesh=_mesh,
    scratch_types=[
        pltpu.VMEM((NCH2, CH2), jnp.int32),
        pltpu.VMEM((NCH2, CH2), jnp.int32),
        pltpu.VMEM((8, CH2), jnp.int32),
        [pltpu.VMEM((CH2, D), jnp.float32)] * NBUF,
        pltpu.VMEM_SHARED((AR, D), jnp.float32),
        [pltpu.SemaphoreType.DMA] * NBUF,
        [pltpu.SemaphoreType.DMA] * NBUF,
    ],
)
def _seg_kernel(hs_hbm, src_hbm, dst_hbm, out_hbm, src_v, dst_v, dloc_v,
                rows, acc_sh, gsem, ssem):
    c = lax.axis_index("c")
    s = lax.axis_index("s")
    base = c * HALF

    pltpu.sync_copy(src_hbm.at[s], src_v)
    pltpu.sync_copy(dst_hbm.at[s], dst_v)

    def _zrow(i, _):
        for j in range(D // 16):
            rows[0][i, pl.ds(j * 16, 16)] = jnp.zeros((16,), jnp.float32)
        return 0
    lax.fori_loop(0, CH2, _zrow, 0)

    zo = s * ART
    for ln in (128, 128, 64, 8):
        pltpu.sync_copy(rows[0].at[pl.ds(0, ln)],
                        acc_sh.at[pl.ds(pl.multiple_of(zo, 8), ln)])
        zo = zo + ln
    plsc.subcore_barrier()

    def _transform(i, row):
        for j in range(CH2 // 16):
            d = dst_v[i, pl.ds(j * 16, 16)]
            l = d - base
            ok = (l >= 0) & (l < HALF)
            dloc_v[row, pl.ds(j * 16, 16)] = jnp.where(ok, l, HALF)

    def _scat_desc(b):
        return pltpu.make_async_copy(
            rows[b], acc_sh.at[dloc_v.at[b]], ssem[b])

    pltpu.async_copy(hs_hbm.at[src_v.at[0]], rows[0], gsem[0])

    def _pair(p, _):
        for b in range(NBUF):
            i = p * NBUF + b
            bn = (b + 1) % NBUF
            _transform(i, b)
            pltpu.make_async_copy(hs_hbm.at[src_v.at[i]], rows[b],
                                  gsem[b]).wait()
            pltpu.async_copy(rows[b], acc_sh.at[dloc_v.at[b]], ssem[b],
                             add=True)

            @pl.when(i + 1 < NCH2)
            def _():
                if b == 0:
                    @pl.when(i >= 1)
                    def _():
                        _scat_desc(bn).wait()
                else:
                    _scat_desc(bn).wait()
                pltpu.async_copy(hs_hbm.at[src_v.at[i + 1]], rows[bn],
                                 gsem[bn])
        return 0
    lax.fori_loop(0, NCH2 // NBUF, _pair, 0)
    for b in range(NBUF):
        _scat_desc(b).wait()

    plsc.subcore_barrier()
    roff = pl.multiple_of(s * WBT, 8)
    ooff = pl.multiple_of(c * HALF + s * WBT, 8)
    pltpu.sync_copy(acc_sh.at[pl.ds(roff, WBT)], out_hbm.at[pl.ds(ooff, WBT)])


def _norm_body(deg_ref, nout_ref, nin_ref):
    deg = deg_ref[...]
    dsrc = deg[0:1] + deg[2:3]
    ddst = deg[1:2] + deg[3:4]
    nout_ref[...] = lax.rsqrt(jnp.clip(dsrc, 1.0, None))
    nin_ref[...] = lax.rsqrt(jnp.clip(ddst, 1.0, None))


_norm_call = pl.pallas_call(
    _norm_body,
    out_shape=[
        jax.ShapeDtypeStruct((1, NH), jnp.float32),
        jax.ShapeDtypeStruct((1, NH), jnp.float32),
    ],
)


def _proj_body(x_ref, wp_ref, bp_ref, nout_ref, h_ref, hs_ref, hg_ref):
    h = jnp.dot(x_ref[...], wp_ref[...],
                preferred_element_type=jnp.float32) + bp_ref[...]
    h_ref[...] = h
    hg_ref[...] = jnp.sum(h, axis=0, keepdims=True)
    hs_ref[...] = h * nout_ref[...]


_proj_call = pl.pallas_call(
    _proj_body,
    out_shape=[
        jax.ShapeDtypeStruct((N, D), jnp.float32),
        jax.ShapeDtypeStruct((N, D), jnp.float32),
        jax.ShapeDtypeStruct((1, D), jnp.float32),
    ],
)


def _layer_body(h_ref, mp_ref, nin_ref, nout_ref, wc_ref, bc_ref, wg_ref,
                bg_ref, hgin_ref, hnew_ref, hsnew_ref, hgout_ref):
    m = mp_ref[...] * nin_ref[...]
    conv = jnp.dot(m, wc_ref[...],
                   preferred_element_type=jnp.float32) + bc_ref[...]
    x = h_ref[...] + conv
    mu = jnp.mean(x, axis=-1, keepdims=True)
    xc = x - mu
    var = jnp.mean(xc * xc, axis=-1, keepdims=True)
    hn = xc * lax.rsqrt(var + 1e-5)
    hnew_ref[...] = hn
    hsnew_ref[...] = hn * nout_ref[...]
    hgi = jnp.sum(hn, axis=0, keepdims=True)
    g = jnp.dot(hgi, wg_ref[...],
                preferred_element_type=jnp.float32) + bg_ref[...]
    hgout_ref[...] = hgin_ref[...] + jnp.where(g >= 0, g, 0.01 * g)


_layer_call = pl.pallas_call(
    _layer_body,
    out_shape=[
        jax.ShapeDtypeStruct((N, D), jnp.float32),
        jax.ShapeDtypeStruct((N, D), jnp.float32),
        jax.ShapeDtypeStruct((1, D), jnp.float32),
    ],
)


def _mlp_body(hg_ref, w0_ref, b0_ref, w1_ref, b1_ref, w2_ref, b2_ref,
              out_ref):
    x = hg_ref[...]
    x = jnp.dot(x, w0_ref[...], preferred_element_type=jnp.float32) + b0_ref[...]
    x = jnp.maximum(x, 0.0)
    x = jnp.dot(x, w1_ref[...], preferred_element_type=jnp.float32) + b1_ref[...]
    x = jnp.maximum(x, 0.0)
    out_ref[...] = jnp.dot(x, w2_ref[...],
                           preferred_element_type=jnp.float32) + b2_ref[...]


_mlp_call = pl.pallas_call(
    _mlp_body,
    out_shape=jax.ShapeDtypeStruct((1, D), jnp.float32),
)


def kernel(node_features, edge_index, Wp, bp, Wc0, bc0, Wc1, bc1, Wc2, bc2,
           Wg0, bg0, Wg1, bg1, Wg2, bg2, Wm0, bm0, Wm1, bm1, Wm2, bm2):
    src = edge_index[0].reshape(NW, NCH, CH)
    dst = edge_index[1].reshape(NW, NCH, CH)
    pad = EP - E
    src16 = jnp.concatenate(
        [edge_index[0], jnp.zeros((pad,), jnp.int32)]).reshape(NS, NCH2, CH2)
    dst16 = jnp.concatenate(
        [edge_index[1], jnp.full((pad,), NP, jnp.int32)]).reshape(NS, NCH2, CH2)

    deg4 = _deg_kernel(src, dst).reshape(2 * NC, NH)

    nout_row, nin_row = _norm_call(deg4)
    nout = nout_row.reshape(NH, 1)[:N]
    nin = nin_row.reshape(NH, 1)[:N]

    h, hs, hg = _proj_call(node_features, Wp, bp.reshape(1, D), nout)

    for Wc, bc, Wg, bg in ((Wc0, bc0, Wg0, bg0),
                           (Wc1, bc1, Wg1, bg1),
                           (Wc2, bc2, Wg2, bg2)):
        mp = _seg_kernel(hs, src16, dst16)[:N]
        h, hs, hg = _layer_call(h, mp, nin, nout, Wc, bc.reshape(1, D),
                                Wg, bg.reshape(1, D), hg)

    return _mlp_call(hg, Wm0, bm0.reshape(1, D), Wm1, bm1.reshape(1, D),
                     Wm2, bm2.reshape(1, D))


# speedup vs baseline: 1.7481x; 1.7481x over previous
import functools

import jax
import jax.numpy as jnp
from jax import lax
from jax.experimental import pallas as pl
from jax.experimental.pallas import tpu as pltpu
from jax.experimental.pallas import tpu_sc as plsc

N = 10000
E = 320000
D = 128

NC = 2
NS = 16
NW = NC * NS
EPT = E // NW
CH = 80
NCH = EPT // CH
NP = 10240
RPT = NP // NS
ZR = 128

NH = 10240
HPT = NH // NS

_mesh = plsc.VectorSubcoreMesh(core_axis_name="c", subcore_axis_name="s")


@functools.partial(
    pl.kernel,
    out_type=jax.ShapeDtypeStruct((2 * NC * NH,), jnp.float32),
    mesh=_mesh,
    scratch_types=[
        pltpu.VMEM((NCH, CH), jnp.int32),
        pltpu.VMEM((NCH, CH), jnp.int32),
        pltpu.VMEM((CH,), jnp.float32),
        pltpu.VMEM((HPT,), jnp.float32),
        pltpu.VMEM_SHARED((NH,), jnp.float32),
        pltpu.VMEM_SHARED((NH,), jnp.float32),
    ],
)
def _deg_kernel(src_hbm, dst_hbm, out_hbm, src_v, dst_v, ones_v, zeros_v,
                hsrc_sh, hdst_sh):
    c = lax.axis_index("c")
    s = lax.axis_index("s")
    wid = s * NC + c

    pltpu.sync_copy(src_hbm.at[wid], src_v)
    pltpu.sync_copy(dst_hbm.at[wid], dst_v)

    for i in range(CH // 16):
        ones_v[pl.ds(i * 16, 16)] = jnp.ones((16,), jnp.float32)

    def _zero(i, _):
        zeros_v[pl.ds(i * 16, 16)] = jnp.zeros((16,), jnp.float32)
        return 0
    lax.fori_loop(0, HPT // 16, _zero, 0)

    hoff = pl.multiple_of(s * HPT, 128)
    pltpu.sync_copy(zeros_v, hsrc_sh.at[pl.ds(hoff, HPT)])
    pltpu.sync_copy(zeros_v, hdst_sh.at[pl.ds(hoff, HPT)])
    plsc.subcore_barrier()

    def _accum(i, _):
        pltpu.sync_copy(ones_v, hsrc_sh.at[src_v.at[i]], add=True)
        pltpu.sync_copy(ones_v, hdst_sh.at[dst_v.at[i]], add=True)
        return 0
    lax.fori_loop(0, NCH, _accum, 0)

    plsc.subcore_barrier()
    osrc = pl.multiple_of(c * (2 * NH) + s * HPT, 128)
    odst = pl.multiple_of(c * (2 * NH) + NH + s * HPT, 128)
    pltpu.sync_copy(hsrc_sh.at[pl.ds(hoff, HPT)], out_hbm.at[pl.ds(osrc, HPT)])
    pltpu.sync_copy(hdst_sh.at[pl.ds(hoff, HPT)], out_hbm.at[pl.ds(odst, HPT)])


HALF = NP // NC
AR = 5248
ART = AR // NS
WBT = HALF // NS
CH2 = 112
EPAD = 20160
EP = EPAD * NS
NCH2 = EPAD // CH2
NBUF = 3


@functools.partial(
    pl.kernel,
    out_type=jax.ShapeDtypeStruct((NP, D), jnp.float32),
    mesh=_mesh,
    scratch_types=[
        pltpu.VMEM((EPAD,), jnp.int32),
        pltpu.VMEM((EPAD,), jnp.int32),
        pltpu.VMEM((8, CH2), jnp.int32),
        [pltpu.VMEM((CH2, D), jnp.float32)] * NBUF,
        pltpu.VMEM_SHARED((AR, D), jnp.float32),
        [pltpu.SemaphoreType.DMA] * NBUF,
        [pltpu.SemaphoreType.DMA] * NBUF,
    ],
)
def _seg_kernel(hs_hbm, src_hbm, dst_hbm, out_hbm, src_v, dst_v, dloc_v,
                rows, acc_sh, gsem, ssem):
    c = lax.axis_index("c")
    s = lax.axis_index("s")
    base = c * HALF

    pltpu.sync_copy(src_hbm.at[s], src_v)
    pltpu.sync_copy(dst_hbm.at[s], dst_v)

    def _zrow(i, _):
        for j in range(D // 16):
            rows[0][i, pl.ds(j * 16, 16)] = jnp.zeros((16,), jnp.float32)
        return 0
    lax.fori_loop(0, CH2, _zrow, 0)

    zo = s * ART
    for ln in (112, 112, 96, 8):
        pltpu.sync_copy(rows[0].at[pl.ds(0, ln)],
                        acc_sh.at[pl.ds(pl.multiple_of(zo, 8), ln)])
        zo = zo + ln
    plsc.subcore_barrier()

    def _transform(i, row):
        for j in range(CH2 // 16):
            d = dst_v[pl.ds(i * CH2 + j * 16, 16)]
            l = d - base
            ok = (l >= 0) & (l < HALF)
            dloc_v[row, pl.ds(j * 16, 16)] = jnp.where(ok, l, HALF)

    def _gat_desc(i, b):
        return pltpu.make_async_copy(
            hs_hbm.at[src_v.at[pl.ds(pl.multiple_of(i * CH2, 8), CH2)]],
            rows[b], gsem[b])

    def _scat_desc(b):
        return pltpu.make_async_copy(
            rows[b], acc_sh.at[dloc_v.at[b]], ssem[b])

    for b in range(NBUF - 1):
        _gat_desc(b, b).start()

    def _group(p, _):
        for b in range(NBUF):
            i = p * NBUF + b
            bn = (b + NBUF - 1) % NBUF
            _transform(i, b)
            _gat_desc(i, b).wait()
            pltpu.async_copy(rows[b], acc_sh.at[dloc_v.at[b]], ssem[b],
                             add=True)

            @pl.when(i + NBUF - 1 < NCH2)
            def _():
                if b == 0:
                    @pl.when(i >= 1)
                    def _():
                        _scat_desc(bn).wait()
                else:
                    _scat_desc(bn).wait()
                _gat_desc(i + NBUF - 1, bn).start()
        return 0
    lax.fori_loop(0, NCH2 // NBUF, _group, 0)
    for b in range(NBUF):
        _scat_desc(b).wait()

    plsc.subcore_barrier()
    roff = pl.multiple_of(s * WBT, 8)
    ooff = pl.multiple_of(c * HALF + s * WBT, 8)
    pltpu.sync_copy(acc_sh.at[pl.ds(roff, WBT)], out_hbm.at[pl.ds(ooff, WBT)])


def _norm_body(deg_ref, nout_ref, nin_ref):
    deg = deg_ref[...]
    dsrc = deg[0:1] + deg[2:3]
    ddst = deg[1:2] + deg[3:4]
    nout_ref[...] = lax.rsqrt(jnp.clip(dsrc, 1.0, None))
    nin_ref[...] = lax.rsqrt(jnp.clip(ddst, 1.0, None))


_norm_call = pl.pallas_call(
    _norm_body,
    out_shape=[
        jax.ShapeDtypeStruct((1, NH), jnp.float32),
        jax.ShapeDtypeStruct((1, NH), jnp.float32),
    ],
)


def _proj_body(x_ref, wp_ref, bp_ref, nout_ref, h_ref, hs_ref, hg_ref):
    h = jnp.dot(x_ref[...], wp_ref[...],
                preferred_element_type=jnp.float32) + bp_ref[...]
    h_ref[...] = h
    hg_ref[...] = jnp.sum(h, axis=0, keepdims=True)
    hs_ref[...] = h * nout_ref[...]


_proj_call = pl.pallas_call(
    _proj_body,
    out_shape=[
        jax.ShapeDtypeStruct((N, D), jnp.float32),
        jax.ShapeDtypeStruct((N, D), jnp.float32),
        jax.ShapeDtypeStruct((1, D), jnp.float32),
    ],
)


def _layer_body(h_ref, mp_ref, nin_ref, nout_ref, wc_ref, bc_ref, wg_ref,
                bg_ref, hgin_ref, hnew_ref, hsnew_ref, hgout_ref):
    m = mp_ref[...] * nin_ref[...]
    conv = jnp.dot(m, wc_ref[...],
                   preferred_element_type=jnp.float32) + bc_ref[...]
    x = h_ref[...] + conv
    mu = jnp.mean(x, axis=-1, keepdims=True)
    xc = x - mu
    var = jnp.mean(xc * xc, axis=-1, keepdims=True)
    hn = xc * lax.rsqrt(var + 1e-5)
    hnew_ref[...] = hn
    hsnew_ref[...] = hn * nout_ref[...]
    hgi = jnp.sum(hn, axis=0, keepdims=True)
    g = jnp.dot(hgi, wg_ref[...],
                preferred_element_type=jnp.float32) + bg_ref[...]
    hgout_ref[...] = hgin_ref[...] + jnp.where(g >= 0, g, 0.01 * g)


_layer_call = pl.pallas_call(
    _layer_body,
    out_shape=[
        jax.ShapeDtypeStruct((N, D), jnp.float32),
        jax.ShapeDtypeStruct((N, D), jnp.float32),
        jax.ShapeDtypeStruct((1, D), jnp.float32),
    ],
)


def _mlp_body(hg_ref, w0_ref, b0_ref, w1_ref, b1_ref, w2_ref, b2_ref,
              out_ref):
    x = hg_ref[...]
    x = jnp.dot(x, w0_ref[...], preferred_element_type=jnp.float32) + b0_ref[...]
    x = jnp.maximum(x, 0.0)
    x = jnp.dot(x, w1_ref[...], preferred_element_type=jnp.float32) + b1_ref[...]
    x = jnp.maximum(x, 0.0)
    out_ref[...] = jnp.dot(x, w2_ref[...],
                           preferred_element_type=jnp.float32) + b2_ref[...]


_mlp_call = pl.pallas_call(
    _mlp_body,
    out_shape=jax.ShapeDtypeStruct((1, D), jnp.float32),
)


def kernel(node_features, edge_index, Wp, bp, Wc0, bc0, Wc1, bc1, Wc2, bc2,
           Wg0, bg0, Wg1, bg1, Wg2, bg2, Wm0, bm0, Wm1, bm1, Wm2, bm2):
    src = edge_index[0].reshape(NW, NCH, CH)
    dst = edge_index[1].reshape(NW, NCH, CH)
    pad = EP - E
    src16 = jnp.concatenate(
        [edge_index[0], jnp.zeros((pad,), jnp.int32)]).reshape(NS, EPAD)
    dst16 = jnp.concatenate(
        [edge_index[1], jnp.full((pad,), NP, jnp.int32)]).reshape(NS, EPAD)

    deg4 = _deg_kernel(src, dst).reshape(2 * NC, NH)

    nout_row, nin_row = _norm_call(deg4)
    nout = nout_row.reshape(NH, 1)[:N]
    nin = nin_row.reshape(NH, 1)[:N]

    h, hs, hg = _proj_call(node_features, Wp, bp.reshape(1, D), nout)

    for Wc, bc, Wg, bg in ((Wc0, bc0, Wg0, bg0),
                           (Wc1, bc1, Wg1, bg1),
                           (Wc2, bc2, Wg2, bg2)):
        mp = _seg_kernel(hs, src16, dst16)[:N]
        h, hs, hg = _layer_call(h, mp, nin, nout, Wc, bc.reshape(1, D),
                                Wg, bg.reshape(1, D), hg)

    return _mlp_call(hg, Wm0, bm0.reshape(1, D), Wm1, bm1.reshape(1, D),
                     Wm2, bm2.reshape(1, D))


# speedup vs baseline: 1.7501x; 1.0011x over previous
import functools

import jax
import jax.numpy as jnp
from jax import lax
from jax.experimental import pallas as pl
from jax.experimental.pallas import tpu as pltpu
from jax.experimental.pallas import tpu_sc as plsc

N = 10000
E = 320000
D = 128

NC = 2
NS = 16
NW = NC * NS
EPT = E // NW
CH = 80
NCH = EPT // CH
NP = 10240
RPT = NP // NS
ZR = 128

NH = 10240
HPT = NH // NS

_mesh = plsc.VectorSubcoreMesh(core_axis_name="c", subcore_axis_name="s")


@functools.partial(
    pl.kernel,
    out_type=jax.ShapeDtypeStruct((2 * NC * NH,), jnp.float32),
    mesh=_mesh,
    scratch_types=[
        pltpu.VMEM((NCH, CH), jnp.int32),
        pltpu.VMEM((NCH, CH), jnp.int32),
        pltpu.VMEM((CH,), jnp.float32),
        pltpu.VMEM((HPT,), jnp.float32),
        pltpu.VMEM_SHARED((NH,), jnp.float32),
        pltpu.VMEM_SHARED((NH,), jnp.float32),
    ],
)
def _deg_kernel(src_hbm, dst_hbm, out_hbm, src_v, dst_v, ones_v, zeros_v,
                hsrc_sh, hdst_sh):
    c = lax.axis_index("c")
    s = lax.axis_index("s")
    wid = s * NC + c

    pltpu.sync_copy(src_hbm.at[wid], src_v)
    pltpu.sync_copy(dst_hbm.at[wid], dst_v)

    for i in range(CH // 16):
        ones_v[pl.ds(i * 16, 16)] = jnp.ones((16,), jnp.float32)

    def _zero(i, _):
        zeros_v[pl.ds(i * 16, 16)] = jnp.zeros((16,), jnp.float32)
        return 0
    lax.fori_loop(0, HPT // 16, _zero, 0)

    hoff = pl.multiple_of(s * HPT, 128)
    pltpu.sync_copy(zeros_v, hsrc_sh.at[pl.ds(hoff, HPT)])
    pltpu.sync_copy(zeros_v, hdst_sh.at[pl.ds(hoff, HPT)])
    plsc.subcore_barrier()

    def _accum(i, _):
        pltpu.sync_copy(ones_v, hsrc_sh.at[src_v.at[i]], add=True)
        pltpu.sync_copy(ones_v, hdst_sh.at[dst_v.at[i]], add=True)
        return 0
    lax.fori_loop(0, NCH, _accum, 0)

    plsc.subcore_barrier()
    osrc = pl.multiple_of(c * (2 * NH) + s * HPT, 128)
    odst = pl.multiple_of(c * (2 * NH) + NH + s * HPT, 128)
    pltpu.sync_copy(hsrc_sh.at[pl.ds(hoff, HPT)], out_hbm.at[pl.ds(osrc, HPT)])
    pltpu.sync_copy(hdst_sh.at[pl.ds(hoff, HPT)], out_hbm.at[pl.ds(odst, HPT)])


HALF = NP // NC
AR = 5248
ART = AR // NS
WBT = HALF // NS
CH2 = 112
EPAD = 20160
EP = EPAD * NS
NCH2 = EPAD // CH2
NBUF = 2


@functools.partial(
    pl.kernel,
    out_type=jax.ShapeDtypeStruct((NP, D), jnp.float32),
    mesh=_mesh,
    scratch_types=[
        pltpu.VMEM((EPAD,), jnp.int32),
        pltpu.VMEM((EPAD,), jnp.int32),
        pltpu.VMEM((8, CH2), jnp.int32),
        [pltpu.VMEM((CH2, D), jnp.float32)] * NBUF,
        pltpu.VMEM_SHARED((AR, D), jnp.float32),
        [pltpu.SemaphoreType.DMA] * NBUF,
    ],
)
def _seg_kernel(hs_hbm, src_hbm, dst_hbm, out_hbm, src_v, dst_v, dloc_v,
                rows, acc_sh, gsem):
    c = lax.axis_index("c")
    s = lax.axis_index("s")
    base = c * HALF

    pltpu.sync_copy(src_hbm.at[s], src_v)
    pltpu.sync_copy(dst_hbm.at[s], dst_v)

    def _zrow(i, _):
        for j in range(D // 16):
            rows[0][i, pl.ds(j * 16, 16)] = jnp.zeros((16,), jnp.float32)
        return 0
    lax.fori_loop(0, CH2, _zrow, 0)

    zo = s * ART
    for ln in (112, 112, 96, 8):
        pltpu.sync_copy(rows[0].at[pl.ds(0, ln)],
                        acc_sh.at[pl.ds(pl.multiple_of(zo, 8), ln)])
        zo = zo + ln
    plsc.subcore_barrier()

    def _transform(i, row):
        for j in range(CH2 // 16):
            d = dst_v[pl.ds(i * CH2 + j * 16, 16)]
            l = d - base
            ok = (l >= 0) & (l < HALF)
            dloc_v[row, pl.ds(j * 16, 16)] = jnp.where(ok, l, HALF)

    def _gat_desc(i, b):
        return pltpu.make_async_copy(
            hs_hbm.at[src_v.at[pl.ds(pl.multiple_of(i * CH2, 8), CH2)]],
            rows[b], gsem[b])

    _gat_desc(0, 0).start()

    def _pair(p, _):
        i0 = p * 2
        i1 = i0 + 1
        _gat_desc(i1, 1).start()
        _transform(i0, 0)
        _gat_desc(i0, 0).wait()
        pltpu.sync_copy(rows[0], acc_sh.at[dloc_v.at[0]], add=True)

        @pl.when(i0 + 2 < NCH2)
        def _():
            _gat_desc(i0 + 2, 0).start()

        _transform(i1, 1)
        _gat_desc(i1, 1).wait()
        pltpu.sync_copy(rows[1], acc_sh.at[dloc_v.at[1]], add=True)
        return 0
    lax.fori_loop(0, NCH2 // 2, _pair, 0)

    plsc.subcore_barrier()
    roff = pl.multiple_of(s * WBT, 8)
    ooff = pl.multiple_of(c * HALF + s * WBT, 8)
    pltpu.sync_copy(acc_sh.at[pl.ds(roff, WBT)], out_hbm.at[pl.ds(ooff, WBT)])


def _norm_body(deg_ref, nout_ref, nin_ref):
    deg = deg_ref[...]
    dsrc = deg[0:1] + deg[2:3]
    ddst = deg[1:2] + deg[3:4]
    nout_ref[...] = lax.rsqrt(jnp.clip(dsrc, 1.0, None))
    nin_ref[...] = lax.rsqrt(jnp.clip(ddst, 1.0, None))


_norm_call = pl.pallas_call(
    _norm_body,
    out_shape=[
        jax.ShapeDtypeStruct((1, NH), jnp.float32),
        jax.ShapeDtypeStruct((1, NH), jnp.float32),
    ],
)


def _proj_body(x_ref, wp_ref, bp_ref, nout_ref, h_ref, hs_ref, hg_ref):
    h = jnp.dot(x_ref[...], wp_ref[...],
                preferred_element_type=jnp.float32) + bp_ref[...]
    h_ref[...] = h
    hg_ref[...] = jnp.sum(h, axis=0, keepdims=True)
    hs_ref[...] = h * nout_ref[...]


_proj_call = pl.pallas_call(
    _proj_body,
    out_shape=[
        jax.ShapeDtypeStruct((N, D), jnp.float32),
        jax.ShapeDtypeStruct((N, D), jnp.float32),
        jax.ShapeDtypeStruct((1, D), jnp.float32),
    ],
)


def _layer_body(h_ref, mp_ref, nin_ref, nout_ref, wc_ref, bc_ref, wg_ref,
                bg_ref, hgin_ref, hnew_ref, hsnew_ref, hgout_ref):
    m = mp_ref[...] * nin_ref[...]
    conv = jnp.dot(m, wc_ref[...],
                   preferred_element_type=jnp.float32) + bc_ref[...]
    x = h_ref[...] + conv
    mu = jnp.mean(x, axis=-1, keepdims=True)
    xc = x - mu
    var = jnp.mean(xc * xc, axis=-1, keepdims=True)
    hn = xc * lax.rsqrt(var + 1e-5)
    hnew_ref[...] = hn
    hsnew_ref[...] = hn * nout_ref[...]
    hgi = jnp.sum(hn, axis=0, keepdims=True)
    g = jnp.dot(hgi, wg_ref[...],
                preferred_element_type=jnp.float32) + bg_ref[...]
    hgout_ref[...] = hgin_ref[...] + jnp.where(g >= 0, g, 0.01 * g)


_layer_call = pl.pallas_call(
    _layer_body,
    out_shape=[
        jax.ShapeDtypeStruct((N, D), jnp.float32),
        jax.ShapeDtypeStruct((N, D), jnp.float32),
        jax.ShapeDtypeStruct((1, D), jnp.float32),
    ],
)


def _mlp_body(hg_ref, w0_ref, b0_ref, w1_ref, b1_ref, w2_ref, b2_ref,
              out_ref):
    x = hg_ref[...]
    x = jnp.dot(x, w0_ref[...], preferred_element_type=jnp.float32) + b0_ref[...]
    x = jnp.maximum(x, 0.0)
    x = jnp.dot(x, w1_ref[...], preferred_element_type=jnp.float32) + b1_ref[...]
    x = jnp.maximum(x, 0.0)
    out_ref[...] = jnp.dot(x, w2_ref[...],
                           preferred_element_type=jnp.float32) + b2_ref[...]


_mlp_call = pl.pallas_call(
    _mlp_body,
    out_shape=jax.ShapeDtypeStruct((1, D), jnp.float32),
)


def kernel(node_features, edge_index, Wp, bp, Wc0, bc0, Wc1, bc1, Wc2, bc2,
           Wg0, bg0, Wg1, bg1, Wg2, bg2, Wm0, bm0, Wm1, bm1, Wm2, bm2):
    src = edge_index[0].reshape(NW, NCH, CH)
    dst = edge_index[1].reshape(NW, NCH, CH)
    pad = EP - E
    src16 = jnp.concatenate(
        [edge_index[0], jnp.zeros((pad,), jnp.int32)]).reshape(NS, EPAD)
    dst16 = jnp.concatenate(
        [edge_index[1], jnp.full((pad,), NP, jnp.int32)]).reshape(NS, EPAD)

    deg4 = _deg_kernel(src, dst).reshape(2 * NC, NH)

    nout_row, nin_row = _norm_call(deg4)
    nout = nout_row.reshape(NH, 1)[:N]
    nin = nin_row.reshape(NH, 1)[:N]

    h, hs, hg = _proj_call(node_features, Wp, bp.reshape(1, D), nout)

    for Wc, bc, Wg, bg in ((Wc0, bc0, Wg0, bg0),
                           (Wc1, bc1, Wg1, bg1),
                           (Wc2, bc2, Wg2, bg2)):
        mp = _seg_kernel(hs, src16, dst16)[:N]
        h, hs, hg = _layer_call(h, mp, nin, nout, Wc, bc.reshape(1, D),
                                Wg, bg.reshape(1, D), hg)

    return _mlp_call(hg, Wm0, bm0.reshape(1, D), Wm1, bm1.reshape(1, D),
                     Wm2, bm2.reshape(1, D))


# speedup vs baseline: 1.7516x; 1.0009x over previous
import functools

import jax
import jax.numpy as jnp
from jax import lax
from jax.experimental import pallas as pl
from jax.experimental.pallas import tpu as pltpu
from jax.experimental.pallas import tpu_sc as plsc

N = 10000
E = 320000
D = 128

NC = 2
NS = 16
NW = NC * NS
EPT = E // NW
CH = 80
NCH = EPT // CH
NP = 10240
RPT = NP // NS
ZR = 128

NH = 10240
HPT = NH // NS

_mesh = plsc.VectorSubcoreMesh(core_axis_name="c", subcore_axis_name="s")


@functools.partial(
    pl.kernel,
    out_type=jax.ShapeDtypeStruct((2 * NC * NH,), jnp.float32),
    mesh=_mesh,
    scratch_types=[
        pltpu.VMEM((NCH, CH), jnp.int32),
        pltpu.VMEM((NCH, CH), jnp.int32),
        pltpu.VMEM((CH,), jnp.float32),
        pltpu.VMEM((HPT,), jnp.float32),
        pltpu.VMEM_SHARED((NH,), jnp.float32),
        pltpu.VMEM_SHARED((NH,), jnp.float32),
    ],
)
def _deg_kernel(src_hbm, dst_hbm, out_hbm, src_v, dst_v, ones_v, zeros_v,
                hsrc_sh, hdst_sh):
    c = lax.axis_index("c")
    s = lax.axis_index("s")
    wid = s * NC + c

    pltpu.sync_copy(src_hbm.at[wid], src_v)
    pltpu.sync_copy(dst_hbm.at[wid], dst_v)

    for i in range(CH // 16):
        ones_v[pl.ds(i * 16, 16)] = jnp.ones((16,), jnp.float32)

    def _zero(i, _):
        zeros_v[pl.ds(i * 16, 16)] = jnp.zeros((16,), jnp.float32)
        return 0
    lax.fori_loop(0, HPT // 16, _zero, 0)

    hoff = pl.multiple_of(s * HPT, 128)
    pltpu.sync_copy(zeros_v, hsrc_sh.at[pl.ds(hoff, HPT)])
    pltpu.sync_copy(zeros_v, hdst_sh.at[pl.ds(hoff, HPT)])
    plsc.subcore_barrier()

    def _accum(i, _):
        pltpu.sync_copy(ones_v, hsrc_sh.at[src_v.at[i]], add=True)
        pltpu.sync_copy(ones_v, hdst_sh.at[dst_v.at[i]], add=True)
        return 0
    lax.fori_loop(0, NCH, _accum, 0)

    plsc.subcore_barrier()
    osrc = pl.multiple_of(c * (2 * NH) + s * HPT, 128)
    odst = pl.multiple_of(c * (2 * NH) + NH + s * HPT, 128)
    pltpu.sync_copy(hsrc_sh.at[pl.ds(hoff, HPT)], out_hbm.at[pl.ds(osrc, HPT)])
    pltpu.sync_copy(hdst_sh.at[pl.ds(hoff, HPT)], out_hbm.at[pl.ds(odst, HPT)])


HALF = NP // NC
AR = 5248
ART = AR // NS
WBT = HALF // NS
CH2 = 112
EPAD = 20160
EP = EPAD * NS
NCH2 = EPAD // CH2
NBUF = 2


@functools.partial(
    pl.kernel,
    out_type=jax.ShapeDtypeStruct((NP, D), jnp.float32),
    mesh=_mesh,
    scratch_types=[
        pltpu.VMEM((NCH2, CH2), jnp.int32),
        pltpu.VMEM((NCH2, CH2), jnp.int32),
        pltpu.VMEM((8, CH2), jnp.int32),
        [pltpu.VMEM((CH2, D), jnp.float32)] * NBUF,
        pltpu.VMEM_SHARED((AR, D), jnp.float32),
        [pltpu.SemaphoreType.DMA] * NBUF,
    ],
)
def _seg_kernel(hs_hbm, src_hbm, dst_hbm, out_hbm, src_v, dst_v, dloc_v,
                rows, acc_sh, gsem):
    c = lax.axis_index("c")
    s = lax.axis_index("s")
    base = c * HALF

    pltpu.sync_copy(src_hbm.at[s], src_v)
    pltpu.sync_copy(dst_hbm.at[s], dst_v)

    def _zrow(i, _):
        for j in range(D // 16):
            rows[0][i, pl.ds(j * 16, 16)] = jnp.zeros((16,), jnp.float32)
        return 0
    lax.fori_loop(0, CH2, _zrow, 0)

    zo = s * ART
    for ln in (112, 112, 96, 8):
        pltpu.sync_copy(rows[0].at[pl.ds(0, ln)],
                        acc_sh.at[pl.ds(pl.multiple_of(zo, 8), ln)])
        zo = zo + ln
    plsc.subcore_barrier()

    def _transform(i, row):
        for j in range(CH2 // 16):
            d = dst_v[i, pl.ds(j * 16, 16)]
            l = d - base
            ok = (l >= 0) & (l < HALF)
            dloc_v[row, pl.ds(j * 16, 16)] = jnp.where(ok, l, HALF)

    def _gat_desc(i, b):
        return pltpu.make_async_copy(hs_hbm.at[src_v.at[i]], rows[b], gsem[b])

    _gat_desc(0, 0).start()

    def _pair(p, _):
        i0 = p * 2
        i1 = i0 + 1
        _gat_desc(i1, 1).start()
        _transform(i0, 0)
        _gat_desc(i0, 0).wait()
        pltpu.sync_copy(rows[0], acc_sh.at[dloc_v.at[0]], add=True)

        @pl.when(i0 + 2 < NCH2)
        def _():
            _gat_desc(i0 + 2, 0).start()

        _transform(i1, 1)
        _gat_desc(i1, 1).wait()
        pltpu.sync_copy(rows[1], acc_sh.at[dloc_v.at[1]], add=True)
        return 0
    lax.fori_loop(0, NCH2 // 2, _pair, 0)

    plsc.subcore_barrier()
    roff = pl.multiple_of(s * WBT, 8)
    ooff = pl.multiple_of(c * HALF + s * WBT, 8)
    pltpu.sync_copy(acc_sh.at[pl.ds(roff, WBT)], out_hbm.at[pl.ds(ooff, WBT)])


def _norm_body(deg_ref, nout_ref, nin_ref):
    deg = deg_ref[...]
    dsrc = deg[0:1] + deg[2:3]
    ddst = deg[1:2] + deg[3:4]
    nout_ref[...] = lax.rsqrt(jnp.clip(dsrc, 1.0, None))
    nin_ref[...] = lax.rsqrt(jnp.clip(ddst, 1.0, None))


_norm_call = pl.pallas_call(
    _norm_body,
    out_shape=[
        jax.ShapeDtypeStruct((1, NH), jnp.float32),
        jax.ShapeDtypeStruct((1, NH), jnp.float32),
    ],
)


def _proj_body(x_ref, wp_ref, bp_ref, nout_ref, h_ref, hs_ref, hg_ref):
    h = jnp.dot(x_ref[...], wp_ref[...],
                preferred_element_type=jnp.float32) + bp_ref[...]
    h_ref[...] = h
    hg_ref[...] = jnp.sum(h, axis=0, keepdims=True)
    hs_ref[...] = h * nout_ref[...]


_proj_call = pl.pallas_call(
    _proj_body,
    out_shape=[
        jax.ShapeDtypeStruct((N, D), jnp.float32),
        jax.ShapeDtypeStruct((N, D), jnp.float32),
        jax.ShapeDtypeStruct((1, D), jnp.float32),
    ],
)


def _layer_body(h_ref, mp_ref, nin_ref, nout_ref, wc_ref, bc_ref, wg_ref,
                bg_ref, hgin_ref, hnew_ref, hsnew_ref, hgout_ref):
    m = mp_ref[...] * nin_ref[...]
    conv = jnp.dot(m, wc_ref[...],
                   preferred_element_type=jnp.float32) + bc_ref[...]
    x = h_ref[...] + conv
    mu = jnp.mean(x, axis=-1, keepdims=True)
    xc = x - mu
    var = jnp.mean(xc * xc, axis=-1, keepdims=True)
    hn = xc * lax.rsqrt(var + 1e-5)
    hnew_ref[...] = hn
    hsnew_ref[...] = hn * nout_ref[...]
    hgi = jnp.sum(hn, axis=0, keepdims=True)
    g = jnp.dot(hgi, wg_ref[...],
                preferred_element_type=jnp.float32) + bg_ref[...]
    hgout_ref[...] = hgin_ref[...] + jnp.where(g >= 0, g, 0.01 * g)


_layer_call = pl.pallas_call(
    _layer_body,
    out_shape=[
        jax.ShapeDtypeStruct((N, D), jnp.float32),
        jax.ShapeDtypeStruct((N, D), jnp.float32),
        jax.ShapeDtypeStruct((1, D), jnp.float32),
    ],
)


def _mlp_body(hg_ref, w0_ref, b0_ref, w1_ref, b1_ref, w2_ref, b2_ref,
              out_ref):
    x = hg_ref[...]
    x = jnp.dot(x, w0_ref[...], preferred_element_type=jnp.float32) + b0_ref[...]
    x = jnp.maximum(x, 0.0)
    x = jnp.dot(x, w1_ref[...], preferred_element_type=jnp.float32) + b1_ref[...]
    x = jnp.maximum(x, 0.0)
    out_ref[...] = jnp.dot(x, w2_ref[...],
                           preferred_element_type=jnp.float32) + b2_ref[...]


_mlp_call = pl.pallas_call(
    _mlp_body,
    out_shape=jax.ShapeDtypeStruct((1, D), jnp.float32),
)


def kernel(node_features, edge_index, Wp, bp, Wc0, bc0, Wc1, bc1, Wc2, bc2,
           Wg0, bg0, Wg1, bg1, Wg2, bg2, Wm0, bm0, Wm1, bm1, Wm2, bm2):
    src = edge_index[0].reshape(NW, NCH, CH)
    dst = edge_index[1].reshape(NW, NCH, CH)
    pad = EP - E
    src16 = jnp.concatenate(
        [edge_index[0], jnp.zeros((pad,), jnp.int32)]).reshape(NS, NCH2, CH2)
    dst16 = jnp.concatenate(
        [edge_index[1], jnp.full((pad,), NP, jnp.int32)]).reshape(NS, NCH2, CH2)

    deg4 = _deg_kernel(src, dst).reshape(2 * NC, NH)

    nout_row, nin_row = _norm_call(deg4)
    nout = nout_row.reshape(NH, 1)[:N]
    nin = nin_row.reshape(NH, 1)[:N]

    h, hs, hg = _proj_call(node_features, Wp, bp.reshape(1, D), nout)

    for Wc, bc, Wg, bg in ((Wc0, bc0, Wg0, bg0),
                           (Wc1, bc1, Wg1, bg1),
                           (Wc2, bc2, Wg2, bg2)):
        mp = _seg_kernel(hs, src16, dst16)[:N]
        h, hs, hg = _layer_call(h, mp, nin, nout, Wc, bc.reshape(1, D),
                                Wg, bg.reshape(1, D), hg)

    return _mlp_call(hg, Wm0, bm0.reshape(1, D), Wm1, bm1.reshape(1, D),
                     Wm2, bm2.reshape(1, D))


# speedup vs baseline: 2.7774x; 1.5856x over previous
import functools

import jax
import jax.numpy as jnp
from jax import lax
from jax.experimental import pallas as pl
from jax.experimental.pallas import tpu as pltpu
from jax.experimental.pallas import tpu_sc as plsc

N = 10000
E = 320000
D = 128

NC = 2
NS = 16
NW = NC * NS
EPT = E // NW
CH = 80
NCH = EPT // CH
NP = 10240
RPT = NP // NS
ZR = 128

NH = 10240
HPT = NH // NS

_mesh = plsc.VectorSubcoreMesh(core_axis_name="c", subcore_axis_name="s")


@functools.partial(
    pl.kernel,
    out_type=jax.ShapeDtypeStruct((2 * NC * NH,), jnp.float32),
    mesh=_mesh,
    scratch_types=[
        pltpu.VMEM((NCH, CH), jnp.int32),
        pltpu.VMEM((NCH, CH), jnp.int32),
        pltpu.VMEM((CH,), jnp.float32),
        pltpu.VMEM((HPT,), jnp.float32),
        pltpu.VMEM_SHARED((NH,), jnp.float32),
        pltpu.VMEM_SHARED((NH,), jnp.float32),
    ],
)
def _deg_kernel(src_hbm, dst_hbm, out_hbm, src_v, dst_v, ones_v, zeros_v,
                hsrc_sh, hdst_sh):
    c = lax.axis_index("c")
    s = lax.axis_index("s")
    wid = s * NC + c

    pltpu.sync_copy(src_hbm.at[wid], src_v)
    pltpu.sync_copy(dst_hbm.at[wid], dst_v)

    for i in range(CH // 16):
        ones_v[pl.ds(i * 16, 16)] = jnp.ones((16,), jnp.float32)

    def _zero(i, _):
        zeros_v[pl.ds(i * 16, 16)] = jnp.zeros((16,), jnp.float32)
        return 0
    lax.fori_loop(0, HPT // 16, _zero, 0)

    hoff = pl.multiple_of(s * HPT, 128)
    pltpu.sync_copy(zeros_v, hsrc_sh.at[pl.ds(hoff, HPT)])
    pltpu.sync_copy(zeros_v, hdst_sh.at[pl.ds(hoff, HPT)])
    plsc.subcore_barrier()

    def _accum(i, _):
        pltpu.sync_copy(ones_v, hsrc_sh.at[src_v.at[i]], add=True)
        pltpu.sync_copy(ones_v, hdst_sh.at[dst_v.at[i]], add=True)
        return 0
    lax.fori_loop(0, NCH, _accum, 0)

    plsc.subcore_barrier()
    osrc = pl.multiple_of(c * (2 * NH) + s * HPT, 128)
    odst = pl.multiple_of(c * (2 * NH) + NH + s * HPT, 128)
    pltpu.sync_copy(hsrc_sh.at[pl.ds(hoff, HPT)], out_hbm.at[pl.ds(osrc, HPT)])
    pltpu.sync_copy(hdst_sh.at[pl.ds(hoff, HPT)], out_hbm.at[pl.ds(odst, HPT)])


HALF = NP // NC
AR = 5248
ART = AR // NS
WBT = HALF // NS
CH2 = 80
EPAD = 20000
EP = EPAD * NS
NCH2 = EPAD // CH2
NBUF = 2


@functools.partial(
    pl.kernel,
    out_type=jax.ShapeDtypeStruct((NP, D), jnp.float32),
    mesh=_mesh,
    scratch_types=[
        pltpu.VMEM((NCH2, CH2), jnp.int32),
        pltpu.VMEM((NCH2, CH2), jnp.int32),
        pltpu.VMEM((8, CH2), jnp.int32),
        [pltpu.VMEM((CH2, D), jnp.float32)] * NBUF,
        pltpu.VMEM_SHARED((AR, D), jnp.float32),
        [pltpu.SemaphoreType.DMA] * NBUF,
    ],
)
def _seg_kernel(hs_hbm, src_hbm, dst_hbm, out_hbm, src_v, dst_v, dloc_v,
                rows, acc_sh, gsem):
    c = lax.axis_index("c")
    s = lax.axis_index("s")
    base = c * HALF

    pltpu.sync_copy(src_hbm.at[s], src_v)
    pltpu.sync_copy(dst_hbm.at[s], dst_v)

    def _zrow(i, _):
        for j in range(D // 16):
            rows[0][i, pl.ds(j * 16, 16)] = jnp.zeros((16,), jnp.float32)
        return 0
    lax.fori_loop(0, CH2, _zrow, 0)

    zo = s * ART
    for ln in (80, 80, 80, 80, 8):
        pltpu.sync_copy(rows[0].at[pl.ds(0, ln)],
                        acc_sh.at[pl.ds(pl.multiple_of(zo, 8), ln)])
        zo = zo + ln
    plsc.subcore_barrier()

    def _transform(i, row):
        for j in range(CH2 // 16):
            d = dst_v[i, pl.ds(j * 16, 16)]
            l = d - base
            ok = (l >= 0) & (l < HALF)
            dloc_v[row, pl.ds(j * 16, 16)] = jnp.where(ok, l, HALF)

    def _gat_desc(i, b):
        return pltpu.make_async_copy(hs_hbm.at[src_v.at[i]], rows[b], gsem[b])

    _gat_desc(0, 0).start()

    def _pair(p, _):
        i0 = p * 2
        i1 = i0 + 1
        _gat_desc(i1, 1).start()
        _transform(i0, 0)
        _gat_desc(i0, 0).wait()
        pltpu.sync_copy(rows[0], acc_sh.at[dloc_v.at[0]], add=True)

        @pl.when(i0 + 2 < NCH2)
        def _():
            _gat_desc(i0 + 2, 0).start()

        _transform(i1, 1)
        _gat_desc(i1, 1).wait()
        pltpu.sync_copy(rows[1], acc_sh.at[dloc_v.at[1]], add=True)
        return 0
    lax.fori_loop(0, NCH2 // 2, _pair, 0)

    plsc.subcore_barrier()
    roff = pl.multiple_of(s * WBT, 8)
    ooff = pl.multiple_of(c * HALF + s * WBT, 8)
    pltpu.sync_copy(acc_sh.at[pl.ds(roff, WBT)], out_hbm.at[pl.ds(ooff, WBT)])


def _norm_body(deg_ref, nout_ref, nin_ref):
    deg = deg_ref[...]
    dsrc = deg[0:1] + deg[2:3]
    ddst = deg[1:2] + deg[3:4]
    nout_ref[...] = lax.rsqrt(jnp.clip(dsrc, 1.0, None))
    nin_ref[...] = lax.rsqrt(jnp.clip(ddst, 1.0, None))


_norm_call = pl.pallas_call(
    _norm_body,
    out_shape=[
        jax.ShapeDtypeStruct((1, NH), jnp.float32),
        jax.ShapeDtypeStruct((1, NH), jnp.float32),
    ],
)


def _proj_body(x_ref, wp_ref, bp_ref, nout_ref, h_ref, hs_ref, hg_ref):
    h = jnp.dot(x_ref[...], wp_ref[...],
                preferred_element_type=jnp.float32) + bp_ref[...]
    h_ref[...] = h
    hg_ref[...] = jnp.sum(h, axis=0, keepdims=True)
    hs_ref[...] = h * nout_ref[...]


_proj_call = pl.pallas_call(
    _proj_body,
    out_shape=[
        jax.ShapeDtypeStruct((N, D), jnp.float32),
        jax.ShapeDtypeStruct((N, D), jnp.float32),
        jax.ShapeDtypeStruct((1, D), jnp.float32),
    ],
)


def _layer_body(h_ref, mp_ref, nin_ref, nout_ref, wc_ref, bc_ref, wg_ref,
                bg_ref, hgin_ref, hnew_ref, hsnew_ref, hgout_ref):
    m = mp_ref[...] * nin_ref[...]
    conv = jnp.dot(m, wc_ref[...],
                   preferred_element_type=jnp.float32) + bc_ref[...]
    x = h_ref[...] + conv
    mu = jnp.mean(x, axis=-1, keepdims=True)
    xc = x - mu
    var = jnp.mean(xc * xc, axis=-1, keepdims=True)
    hn = xc * lax.rsqrt(var + 1e-5)
    hnew_ref[...] = hn
    hsnew_ref[...] = hn * nout_ref[...]
    hgi = jnp.sum(hn, axis=0, keepdims=True)
    g = jnp.dot(hgi, wg_ref[...],
                preferred_element_type=jnp.float32) + bg_ref[...]
    hgout_ref[...] = hgin_ref[...] + jnp.where(g >= 0, g, 0.01 * g)


_layer_call = pl.pallas_call(
    _layer_body,
    out_shape=[
        jax.ShapeDtypeStruct((N, D), jnp.float32),
        jax.ShapeDtypeStruct((N, D), jnp.float32),
        jax.ShapeDtypeStruct((1, D), jnp.float32),
    ],
)


def _mlp_body(hg_ref, w0_ref, b0_ref, w1_ref, b1_ref, w2_ref, b2_ref,
              out_ref):
    x = hg_ref[...]
    x = jnp.dot(x, w0_ref[...], preferred_element_type=jnp.float32) + b0_ref[...]
    x = jnp.maximum(x, 0.0)
    x = jnp.dot(x, w1_ref[...], preferred_element_type=jnp.float32) + b1_ref[...]
    x = jnp.maximum(x, 0.0)
    out_ref[...] = jnp.dot(x, w2_ref[...],
                           preferred_element_type=jnp.float32) + b2_ref[...]


_mlp_call = pl.pallas_call(
    _mlp_body,
    out_shape=jax.ShapeDtypeStruct((1, D), jnp.float32),
)


def kernel(node_features, edge_index, Wp, bp, Wc0, bc0, Wc1, bc1, Wc2, bc2,
           Wg0, bg0, Wg1, bg1, Wg2, bg2, Wm0, bm0, Wm1, bm1, Wm2, bm2):
    src = edge_index[0].reshape(NW, NCH, CH)
    dst = edge_index[1].reshape(NW, NCH, CH)
    src16 = edge_index[0].reshape(NS, NCH2, CH2)
    dst16 = edge_index[1].reshape(NS, NCH2, CH2)

    deg4 = _deg_kernel(src, dst).reshape(2 * NC, NH)

    nout_row, nin_row = _norm_call(deg4)
    nout = nout_row.reshape(NH, 1)[:N]
    nin = nin_row.reshape(NH, 1)[:N]

    h, hs, hg = _proj_call(node_features, Wp, bp.reshape(1, D), nout)

    for Wc, bc, Wg, bg in ((Wc0, bc0, Wg0, bg0),
                           (Wc1, bc1, Wg1, bg1),
                           (Wc2, bc2, Wg2, bg2)):
        mp = _seg_kernel(hs, src16, dst16)[:N]
        h, hs, hg = _layer_call(h, mp, nin, nout, Wc, bc.reshape(1, D),
                                Wg, bg.reshape(1, D), hg)

    return _mlp_call(hg, Wm0, bm0.reshape(1, D), Wm1, bm1.reshape(1, D),
                     Wm2, bm2.reshape(1, D))


# speedup vs baseline: 3.9942x; 1.4381x over previous
import functools

import jax
import jax.numpy as jnp
from jax import lax
from jax.experimental import pallas as pl
from jax.experimental.pallas import tpu as pltpu
from jax.experimental.pallas import tpu_sc as plsc

N = 10000
E = 320000
D = 128

NC = 2
NS = 16
NW = NC * NS
EPT = E // NW
CH = 80
NCH = EPT // CH
NP = 10240
RPT = NP // NS
ZR = 128

NH = 10240
HPT = NH // NS

_mesh = plsc.VectorSubcoreMesh(core_axis_name="c", subcore_axis_name="s")


@functools.partial(
    pl.kernel,
    out_type=jax.ShapeDtypeStruct((2 * NC * NH,), jnp.float32),
    mesh=_mesh,
    scratch_types=[
        pltpu.VMEM((NCH, CH), jnp.int32),
        pltpu.VMEM((NCH, CH), jnp.int32),
        pltpu.VMEM((CH,), jnp.float32),
        pltpu.VMEM((HPT,), jnp.float32),
        pltpu.VMEM_SHARED((NH,), jnp.float32),
        pltpu.VMEM_SHARED((NH,), jnp.float32),
    ],
)
def _deg_kernel(src_hbm, dst_hbm, out_hbm, src_v, dst_v, ones_v, zeros_v,
                hsrc_sh, hdst_sh):
    c = lax.axis_index("c")
    s = lax.axis_index("s")
    wid = s * NC + c

    pltpu.sync_copy(src_hbm.at[wid], src_v)
    pltpu.sync_copy(dst_hbm.at[wid], dst_v)

    for i in range(CH // 16):
        ones_v[pl.ds(i * 16, 16)] = jnp.ones((16,), jnp.float32)

    def _zero(i, _):
        zeros_v[pl.ds(i * 16, 16)] = jnp.zeros((16,), jnp.float32)
        return 0
    lax.fori_loop(0, HPT // 16, _zero, 0)

    hoff = pl.multiple_of(s * HPT, 128)
    pltpu.sync_copy(zeros_v, hsrc_sh.at[pl.ds(hoff, HPT)])
    pltpu.sync_copy(zeros_v, hdst_sh.at[pl.ds(hoff, HPT)])
    plsc.subcore_barrier()

    def _accum(i, _):
        pltpu.sync_copy(ones_v, hsrc_sh.at[src_v.at[i]], add=True)
        pltpu.sync_copy(ones_v, hdst_sh.at[dst_v.at[i]], add=True)
        return 0
    lax.fori_loop(0, NCH, _accum, 0)

    plsc.subcore_barrier()
    osrc = pl.multiple_of(c * (2 * NH) + s * HPT, 128)
    odst = pl.multiple_of(c * (2 * NH) + NH + s * HPT, 128)
    pltpu.sync_copy(hsrc_sh.at[pl.ds(hoff, HPT)], out_hbm.at[pl.ds(osrc, HPT)])
    pltpu.sync_copy(hdst_sh.at[pl.ds(hoff, HPT)], out_hbm.at[pl.ds(odst, HPT)])


HALF = NP // NC
AR = 5248
ART = AR // NS
WBT = HALF // NS
CH2 = 80
EPT2 = E // NS
BCH = 256
PAGE = 96
NBUF = 4


@functools.partial(
    pl.kernel,
    out_type=[
        jax.ShapeDtypeStruct((NC, NS, BCH, CH2), jnp.int32),
        jax.ShapeDtypeStruct((NC, NS, BCH, CH2), jnp.int32),
        jax.ShapeDtypeStruct((NC * NS * 16,), jnp.int32),
    ],
    mesh=_mesh,
    compiler_params=pltpu.CompilerParams(needs_layout_passes=False),
    scratch_types=[
        pltpu.VMEM((EPT2,), jnp.int32),
        pltpu.VMEM((EPT2,), jnp.int32),
        pltpu.VMEM((BCH, CH2), jnp.int32),
        pltpu.VMEM((BCH, CH2), jnp.int32),
        pltpu.VMEM((16,), jnp.int32),
    ],
)
def _bucket_kernel(src_hbm, dst_hbm, srcb_hbm, dlocb_hbm, cnt_hbm,
                   src_v, dst_v, srcb_v, dlocb_v, cnt_v):
    c = lax.axis_index("c")
    s = lax.axis_index("s")
    base = c * HALF

    pltpu.sync_copy(src_hbm.at[s], src_v)
    pltpu.sync_copy(dst_hbm.at[s], dst_v)

    def _scan(g, cnt):
        for u in range(2):
            d = dst_v[pl.ds((g * 2 + u) * 16, 16)]
            sv = src_v[pl.ds((g * 2 + u) * 16, 16)]
            l = d - base
            ok = (l >= 0) & (l < HALF)
            oki = jnp.where(ok, jnp.int32(1), jnp.int32(0))
            cum = plsc.cumsum(oki)
            pos = cnt + cum - 1
            r = pos // CH2
            col = pos % CH2
            plsc.store_scatter(srcb_v, [r, col], sv, mask=ok)
            plsc.store_scatter(dlocb_v, [r, col], l, mask=ok)
            cnt = cnt + cum[15]
        return cnt
    cnt = lax.fori_loop(0, EPT2 // 32, _scan, jnp.int32(0))

    for k in range(CH2 // 16):
        pos = cnt + k * 16 + lax.iota(jnp.int32, 16)
        r = pos // CH2
        col = pos % CH2
        plsc.store_scatter(srcb_v, [r, col], jnp.zeros((16,), jnp.int32))
        plsc.store_scatter(dlocb_v, [r, col],
                           jnp.full((16,), HALF, jnp.int32))

    nch = (cnt + CH2 - 1) // CH2
    cnt_v[...] = jnp.full((16,), nch, jnp.int32)
    pltpu.sync_copy(srcb_v, srcb_hbm.at[c, s])
    pltpu.sync_copy(dlocb_v, dlocb_hbm.at[c, s])
    coff = pl.multiple_of((c * NS + s) * 16, 16)
    pltpu.sync_copy(cnt_v, cnt_hbm.at[pl.ds(coff, 16)])


@functools.partial(
    pl.kernel,
    out_type=jax.ShapeDtypeStruct((NP, D), jnp.float32),
    mesh=_mesh,
    scratch_types=[
        pltpu.VMEM((PAGE, CH2), jnp.int32),
        pltpu.VMEM((PAGE, CH2), jnp.int32),
        pltpu.VMEM((16,), jnp.int32),
        [pltpu.VMEM((CH2, D), jnp.float32)] * NBUF,
        pltpu.VMEM_SHARED((AR, D), jnp.float32),
        [pltpu.SemaphoreType.DMA] * NBUF,
        [pltpu.SemaphoreType.DMA] * NBUF,
    ],
)
def _seg_kernel(hs_hbm, srcb_hbm, dlocb_hbm, cnt_hbm, out_hbm, src_v, dloc_v,
                cnt_v, rows, acc_sh, gsem, ssem):
    c = lax.axis_index("c")
    s = lax.axis_index("s")

    coff = pl.multiple_of((c * NS + s) * 16, 16)
    pltpu.sync_copy(cnt_hbm.at[pl.ds(coff, 16)], cnt_v)
    nch = jnp.maximum(cnt_v[...][0], 1)

    def _zrow(i, _):
        for j in range(D // 16):
            rows[0][i, pl.ds(j * 16, 16)] = jnp.zeros((16,), jnp.float32)
        return 0
    lax.fori_loop(0, CH2, _zrow, 0)

    zo = s * ART
    for ln in (80, 80, 80, 80, 8):
        pltpu.sync_copy(rows[0].at[pl.ds(0, ln)],
                        acc_sh.at[pl.ds(pl.multiple_of(zo, 8), ln)])
        zo = zo + ln
    plsc.subcore_barrier()

    def _gat_desc(i, b):
        return pltpu.make_async_copy(hs_hbm.at[src_v.at[i]], rows[b], gsem[b])

    def _scat_desc(i, b):
        return pltpu.make_async_copy(
            rows[b], acc_sh.at[dloc_v.at[i]], ssem[b])

    def _page_body(pg, m):
        pltpu.sync_copy(srcb_hbm.at[c, s, pl.ds(pg * PAGE, PAGE)], src_v)
        pltpu.sync_copy(dlocb_hbm.at[c, s, pl.ds(pg * PAGE, PAGE)], dloc_v)
        for b in range(NBUF - 1):
            @pl.when(b < m)
            def _():
                _gat_desc(b, b).start()

        def _group(p, _):
            for b in range(NBUF):
                i = p * NBUF + b
                bn = (b + NBUF - 1) % NBUF

                @pl.when(i < m)
                def _():
                    _gat_desc(i, b).wait()
                    pltpu.async_copy(rows[b], acc_sh.at[dloc_v.at[i]],
                                     ssem[b], add=True)

                @pl.when((i + NBUF - 1 < m) & (i >= 1))
                def _():
                    _scat_desc(i - 1, bn).wait()

                @pl.when(i + NBUF - 1 < m)
                def _():
                    _gat_desc(i + NBUF - 1, bn).start()
            return 0
        lax.fori_loop(0, (m + NBUF - 1) // NBUF, _group, 0)
        for b in range(NBUF):
            @pl.when(b < m)
            def _():
                _scat_desc(0, b).wait()

    m0 = jnp.minimum(nch, PAGE)
    _page_body(0, m0)
    m1 = nch - PAGE

    @pl.when(m1 > 0)
    def _():
        _page_body(1, m1)

    plsc.subcore_barrier()
    roff = pl.multiple_of(s * WBT, 8)
    ooff = pl.multiple_of(c * HALF + s * WBT, 8)
    pltpu.sync_copy(acc_sh.at[pl.ds(roff, WBT)], out_hbm.at[pl.ds(ooff, WBT)])


def _norm_body(deg_ref, nout_ref, nin_ref):
    deg = deg_ref[...]
    dsrc = deg[0:1] + deg[2:3]
    ddst = deg[1:2] + deg[3:4]
    nout_ref[...] = lax.rsqrt(jnp.clip(dsrc, 1.0, None))
    nin_ref[...] = lax.rsqrt(jnp.clip(ddst, 1.0, None))


_norm_call = pl.pallas_call(
    _norm_body,
    out_shape=[
        jax.ShapeDtypeStruct((1, NH), jnp.float32),
        jax.ShapeDtypeStruct((1, NH), jnp.float32),
    ],
)


def _proj_body(x_ref, wp_ref, bp_ref, nout_ref, h_ref, hs_ref, hg_ref):
    h = jnp.dot(x_ref[...], wp_ref[...],
                preferred_element_type=jnp.float32) + bp_ref[...]
    h_ref[...] = h
    hg_ref[...] = jnp.sum(h, axis=0, keepdims=True)
    hs_ref[...] = h * nout_ref[...]


_proj_call = pl.pallas_call(
    _proj_body,
    out_shape=[
        jax.ShapeDtypeStruct((N, D), jnp.float32),
        jax.ShapeDtypeStruct((N, D), jnp.float32),
        jax.ShapeDtypeStruct((1, D), jnp.float32),
    ],
)


def _layer_core(h_ref, mp_ref, nin_ref, nout_ref, wc_ref, bc_ref, wg_ref,
                bg_ref, hgin_ref):
    m = mp_ref[...] * nin_ref[...]
    conv = jnp.dot(m, wc_ref[...],
                   preferred_element_type=jnp.float32) + bc_ref[...]
    x = h_ref[...] + conv
    mu = jnp.mean(x, axis=-1, keepdims=True)
    xc = x - mu
    var = jnp.mean(xc * xc, axis=-1, keepdims=True)
    hn = xc * lax.rsqrt(var + 1e-5)
    hgi = jnp.sum(hn, axis=0, keepdims=True)
    g = jnp.dot(hgi, wg_ref[...],
                preferred_element_type=jnp.float32) + bg_ref[...]
    hg = hgin_ref[...] + jnp.where(g >= 0, g, 0.01 * g)
    return hn, hg


def _layer_body(h_ref, mp_ref, nin_ref, nout_ref, wc_ref, bc_ref, wg_ref,
                bg_ref, hgin_ref, hnew_ref, hsnew_ref, hgout_ref):
    hn, hg = _layer_core(h_ref, mp_ref, nin_ref, nout_ref, wc_ref, bc_ref,
                         wg_ref, bg_ref, hgin_ref)
    hnew_ref[...] = hn
    hsnew_ref[...] = hn * nout_ref[...]
    hgout_ref[...] = hg


def _last_body(h_ref, mp_ref, nin_ref, nout_ref, wc_ref, bc_ref, wg_ref,
               bg_ref, hgin_ref, w0_ref, b0_ref, w1_ref, b1_ref, w2_ref,
               b2_ref, out_ref):
    _, hg = _layer_core(h_ref, mp_ref, nin_ref, nout_ref, wc_ref, bc_ref,
                        wg_ref, bg_ref, hgin_ref)
    x = jnp.dot(hg, w0_ref[...], preferred_element_type=jnp.float32) + b0_ref[...]
    x = jnp.maximum(x, 0.0)
    x = jnp.dot(x, w1_ref[...], preferred_element_type=jnp.float32) + b1_ref[...]
    x = jnp.maximum(x, 0.0)
    out_ref[...] = jnp.dot(x, w2_ref[...],
                           preferred_element_type=jnp.float32) + b2_ref[...]


_last_call = pl.pallas_call(
    _last_body,
    out_shape=jax.ShapeDtypeStruct((1, D), jnp.float32),
)


_layer_call = pl.pallas_call(
    _layer_body,
    out_shape=[
        jax.ShapeDtypeStruct((N, D), jnp.float32),
        jax.ShapeDtypeStruct((N, D), jnp.float32),
        jax.ShapeDtypeStruct((1, D), jnp.float32),
    ],
)


def _mlp_body(hg_ref, w0_ref, b0_ref, w1_ref, b1_ref, w2_ref, b2_ref,
              out_ref):
    x = hg_ref[...]
    x = jnp.dot(x, w0_ref[...], preferred_element_type=jnp.float32) + b0_ref[...]
    x = jnp.maximum(x, 0.0)
    x = jnp.dot(x, w1_ref[...], preferred_element_type=jnp.float32) + b1_ref[...]
    x = jnp.maximum(x, 0.0)
    out_ref[...] = jnp.dot(x, w2_ref[...],
                           preferred_element_type=jnp.float32) + b2_ref[...]


_mlp_call = pl.pallas_call(
    _mlp_body,
    out_shape=jax.ShapeDtypeStruct((1, D), jnp.float32),
)


def kernel(node_features, edge_index, Wp, bp, Wc0, bc0, Wc1, bc1, Wc2, bc2,
           Wg0, bg0, Wg1, bg1, Wg2, bg2, Wm0, bm0, Wm1, bm1, Wm2, bm2):
    src = edge_index[0].reshape(NW, NCH, CH)
    dst = edge_index[1].reshape(NW, NCH, CH)
    src16 = edge_index[0].reshape(NS, EPT2)
    dst16 = edge_index[1].reshape(NS, EPT2)

    deg4 = _deg_kernel(src, dst).reshape(2 * NC, NH)

    nout_row, nin_row = _norm_call(deg4)
    nout = nout_row.reshape(NH, 1)[:N]
    nin = nin_row.reshape(NH, 1)[:N]

    srcb, dlocb, cnts = _bucket_kernel(src16, dst16)

    h, hs, hg = _proj_call(node_features, Wp, bp.reshape(1, D), nout)

    for Wc, bc, Wg, bg in ((Wc0, bc0, Wg0, bg0),
                           (Wc1, bc1, Wg1, bg1)):
        mp = _seg_kernel(hs, srcb, dlocb, cnts)[:N]
        h, hs, hg = _layer_call(h, mp, nin, nout, Wc, bc.reshape(1, D),
                                Wg, bg.reshape(1, D), hg)

    mp = _seg_kernel(hs, srcb, dlocb, cnts)[:N]
    return _last_call(h, mp, nin, nout, Wc2, bc2.reshape(1, D),
                      Wg2, bg2.reshape(1, D), hg,
                      Wm0, bm0.reshape(1, D), Wm1, bm1.reshape(1, D),
                      Wm2, bm2.reshape(1, D))


# speedup vs baseline: 4.0260x; 1.0080x over previous
import functools

import jax
import jax.numpy as jnp
from jax import lax
from jax.experimental import pallas as pl
from jax.experimental.pallas import tpu as pltpu
from jax.experimental.pallas import tpu_sc as plsc

N = 10000
E = 320000
D = 128

NC = 2
NS = 16
NW = NC * NS
EPT = E // NW
CH = 80
NCH = EPT // CH
NP = 10240
RPT = NP // NS
ZR = 128

NH = 10240
HPT = NH // NS

_mesh = plsc.VectorSubcoreMesh(core_axis_name="c", subcore_axis_name="s")


@functools.partial(
    pl.kernel,
    out_type=jax.ShapeDtypeStruct((2 * NC * NH,), jnp.float32),
    mesh=_mesh,
    scratch_types=[
        pltpu.VMEM((NCH, CH), jnp.int32),
        pltpu.VMEM((NCH, CH), jnp.int32),
        pltpu.VMEM((CH,), jnp.float32),
        pltpu.VMEM((HPT,), jnp.float32),
        pltpu.VMEM_SHARED((NH,), jnp.float32),
        pltpu.VMEM_SHARED((NH,), jnp.float32),
    ],
)
def _deg_kernel(src_hbm, dst_hbm, out_hbm, src_v, dst_v, ones_v, zeros_v,
                hsrc_sh, hdst_sh):
    c = lax.axis_index("c")
    s = lax.axis_index("s")
    wid = s * NC + c

    pltpu.sync_copy(src_hbm.at[wid], src_v)
    pltpu.sync_copy(dst_hbm.at[wid], dst_v)

    for i in range(CH // 16):
        ones_v[pl.ds(i * 16, 16)] = jnp.ones((16,), jnp.float32)

    def _zero(i, _):
        zeros_v[pl.ds(i * 16, 16)] = jnp.zeros((16,), jnp.float32)
        return 0
    lax.fori_loop(0, HPT // 16, _zero, 0)

    hoff = pl.multiple_of(s * HPT, 128)
    pltpu.sync_copy(zeros_v, hsrc_sh.at[pl.ds(hoff, HPT)])
    pltpu.sync_copy(zeros_v, hdst_sh.at[pl.ds(hoff, HPT)])
    plsc.subcore_barrier()

    def _accum(i, _):
        pltpu.sync_copy(ones_v, hsrc_sh.at[src_v.at[i]], add=True)
        pltpu.sync_copy(ones_v, hdst_sh.at[dst_v.at[i]], add=True)
        return 0
    lax.fori_loop(0, NCH, _accum, 0)

    plsc.subcore_barrier()
    osrc = pl.multiple_of(c * (2 * NH) + s * HPT, 128)
    odst = pl.multiple_of(c * (2 * NH) + NH + s * HPT, 128)
    pltpu.sync_copy(hsrc_sh.at[pl.ds(hoff, HPT)], out_hbm.at[pl.ds(osrc, HPT)])
    pltpu.sync_copy(hdst_sh.at[pl.ds(hoff, HPT)], out_hbm.at[pl.ds(odst, HPT)])


HALF = NP // NC
AR = 5248
ART = AR // NS
WBT = HALF // NS
CH2 = 80
EPT2 = E // NS
BCH = 256
PAGE = 128
NBUF = 4


@functools.partial(
    pl.kernel,
    out_type=[
        jax.ShapeDtypeStruct((NC, NS, BCH, CH2), jnp.int32),
        jax.ShapeDtypeStruct((NC, NS, BCH, CH2), jnp.int32),
        jax.ShapeDtypeStruct((NC * NS * 16,), jnp.int32),
    ],
    mesh=_mesh,
    compiler_params=pltpu.CompilerParams(needs_layout_passes=False),
    scratch_types=[
        pltpu.VMEM((EPT2,), jnp.int32),
        pltpu.VMEM((EPT2,), jnp.int32),
        pltpu.VMEM((BCH, CH2), jnp.int32),
        pltpu.VMEM((BCH, CH2), jnp.int32),
        pltpu.VMEM((16,), jnp.int32),
    ],
)
def _bucket_kernel(src_hbm, dst_hbm, srcb_hbm, dlocb_hbm, cnt_hbm,
                   src_v, dst_v, srcb_v, dlocb_v, cnt_v):
    c = lax.axis_index("c")
    s = lax.axis_index("s")
    base = c * HALF

    pltpu.sync_copy(src_hbm.at[s], src_v)
    pltpu.sync_copy(dst_hbm.at[s], dst_v)

    def _scan(g, cnt):
        for u in range(2):
            d = dst_v[pl.ds((g * 2 + u) * 16, 16)]
            sv = src_v[pl.ds((g * 2 + u) * 16, 16)]
            l = d - base
            ok = (l >= 0) & (l < HALF)
            oki = jnp.where(ok, jnp.int32(1), jnp.int32(0))
            cum = plsc.cumsum(oki)
            pos = cnt + cum - 1
            r = pos // CH2
            col = pos % CH2
            plsc.store_scatter(srcb_v, [r, col], sv, mask=ok)
            plsc.store_scatter(dlocb_v, [r, col], l, mask=ok)
            cnt = cnt + cum[15]
        return cnt
    cnt = lax.fori_loop(0, EPT2 // 32, _scan, jnp.int32(0))

    for k in range(CH2 // 16):
        pos = cnt + k * 16 + lax.iota(jnp.int32, 16)
        r = pos // CH2
        col = pos % CH2
        plsc.store_scatter(srcb_v, [r, col], jnp.zeros((16,), jnp.int32))
        plsc.store_scatter(dlocb_v, [r, col],
                           jnp.full((16,), HALF, jnp.int32))

    nch = (cnt + CH2 - 1) // CH2
    cnt_v[...] = jnp.full((16,), nch, jnp.int32)
    pltpu.sync_copy(srcb_v, srcb_hbm.at[c, s])
    pltpu.sync_copy(dlocb_v, dlocb_hbm.at[c, s])
    coff = pl.multiple_of((c * NS + s) * 16, 16)
    pltpu.sync_copy(cnt_v, cnt_hbm.at[pl.ds(coff, 16)])


@functools.partial(
    pl.kernel,
    out_type=jax.ShapeDtypeStruct((NP, D), jnp.float32),
    mesh=_mesh,
    scratch_types=[
        pltpu.VMEM((PAGE, CH2), jnp.int32),
        pltpu.VMEM((PAGE, CH2), jnp.int32),
        pltpu.VMEM((16,), jnp.int32),
        [pltpu.VMEM((CH2, D), jnp.float32)] * NBUF,
        pltpu.VMEM_SHARED((AR, D), jnp.float32),
        [pltpu.SemaphoreType.DMA] * NBUF,
        [pltpu.SemaphoreType.DMA] * NBUF,
    ],
)
def _seg_kernel(hs_hbm, srcb_hbm, dlocb_hbm, cnt_hbm, out_hbm, src_v, dloc_v,
                cnt_v, rows, acc_sh, gsem, ssem):
    c = lax.axis_index("c")
    s = lax.axis_index("s")

    coff = pl.multiple_of((c * NS + s) * 16, 16)
    pltpu.sync_copy(cnt_hbm.at[pl.ds(coff, 16)], cnt_v)
    nch = jnp.maximum(cnt_v[...][0], 1)

    def _zrow(i, _):
        for j in range(D // 16):
            rows[0][i, pl.ds(j * 16, 16)] = jnp.zeros((16,), jnp.float32)
        return 0
    lax.fori_loop(0, CH2, _zrow, 0)

    zo = s * ART
    for ln in (80, 80, 80, 80, 8):
        pltpu.sync_copy(rows[0].at[pl.ds(0, ln)],
                        acc_sh.at[pl.ds(pl.multiple_of(zo, 8), ln)])
        zo = zo + ln
    plsc.subcore_barrier()

    def _gat_desc(i, b):
        return pltpu.make_async_copy(hs_hbm.at[src_v.at[i]], rows[b], gsem[b])

    def _scat_desc(i, b):
        return pltpu.make_async_copy(
            rows[b], acc_sh.at[dloc_v.at[i]], ssem[b])

    def _page_body(pg, m):
        pltpu.sync_copy(srcb_hbm.at[c, s, pl.ds(pg * PAGE, PAGE)], src_v)
        pltpu.sync_copy(dlocb_hbm.at[c, s, pl.ds(pg * PAGE, PAGE)], dloc_v)
        for b in range(NBUF - 1):
            @pl.when(b < m)
            def _():
                _gat_desc(b, b).start()

        def _group(p, _):
            for b in range(NBUF):
                i = p * NBUF + b
                bn = (b + NBUF - 1) % NBUF

                @pl.when(i < m)
                def _():
                    _gat_desc(i, b).wait()
                    pltpu.async_copy(rows[b], acc_sh.at[dloc_v.at[i]],
                                     ssem[b], add=True)

                @pl.when((i + NBUF - 1 < m) & (i >= 1))
                def _():
                    _scat_desc(i - 1, bn).wait()

                @pl.when(i + NBUF - 1 < m)
                def _():
                    _gat_desc(i + NBUF - 1, bn).start()
            return 0
        lax.fori_loop(0, (m + NBUF - 1) // NBUF, _group, 0)
        for b in range(NBUF):
            @pl.when(b < m)
            def _():
                _scat_desc(0, b).wait()

    m0 = jnp.minimum(nch, PAGE)
    _page_body(0, m0)
    m1 = nch - PAGE

    @pl.when(m1 > 0)
    def _():
        _page_body(1, m1)

    plsc.subcore_barrier()
    roff = pl.multiple_of(s * WBT, 8)
    ooff = pl.multiple_of(c * HALF + s * WBT, 8)
    pltpu.sync_copy(acc_sh.at[pl.ds(roff, WBT)], out_hbm.at[pl.ds(ooff, WBT)])


def _norm_body(deg_ref, nout_ref, nin_ref):
    deg = deg_ref[...]
    dsrc = deg[0:1] + deg[2:3]
    ddst = deg[1:2] + deg[3:4]
    nout_ref[...] = lax.rsqrt(jnp.clip(dsrc, 1.0, None))
    nin_ref[...] = lax.rsqrt(jnp.clip(ddst, 1.0, None))


_norm_call = pl.pallas_call(
    _norm_body,
    out_shape=[
        jax.ShapeDtypeStruct((1, NH), jnp.float32),
        jax.ShapeDtypeStruct((1, NH), jnp.float32),
    ],
)


def _proj_body(x_ref, wp_ref, bp_ref, nout_ref, h_ref, hs_ref, hg_ref):
    h = jnp.dot(x_ref[...], wp_ref[...],
                preferred_element_type=jnp.float32) + bp_ref[...]
    h_ref[...] = h
    hg_ref[...] = jnp.sum(h, axis=0, keepdims=True)
    hs_ref[...] = h * nout_ref[...]


_proj_call = pl.pallas_call(
    _proj_body,
    out_shape=[
        jax.ShapeDtypeStruct((N, D), jnp.float32),
        jax.ShapeDtypeStruct((N, D), jnp.float32),
        jax.ShapeDtypeStruct((1, D), jnp.float32),
    ],
)


def _layer_core(h_ref, mp_ref, nin_ref, nout_ref, wc_ref, bc_ref, wg_ref,
                bg_ref, hgin_ref):
    m = mp_ref[...] * nin_ref[...]
    conv = jnp.dot(m, wc_ref[...],
                   preferred_element_type=jnp.float32) + bc_ref[...]
    x = h_ref[...] + conv
    mu = jnp.mean(x, axis=-1, keepdims=True)
    xc = x - mu
    var = jnp.mean(xc * xc, axis=-1, keepdims=True)
    hn = xc * lax.rsqrt(var + 1e-5)
    hgi = jnp.sum(hn, axis=0, keepdims=True)
    g = jnp.dot(hgi, wg_ref[...],
                preferred_element_type=jnp.float32) + bg_ref[...]
    hg = hgin_ref[...] + jnp.where(g >= 0, g, 0.01 * g)
    return hn, hg


def _layer_body(h_ref, mp_ref, nin_ref, nout_ref, wc_ref, bc_ref, wg_ref,
                bg_ref, hgin_ref, hnew_ref, hsnew_ref, hgout_ref):
    hn, hg = _layer_core(h_ref, mp_ref, nin_ref, nout_ref, wc_ref, bc_ref,
                         wg_ref, bg_ref, hgin_ref)
    hnew_ref[...] = hn
    hsnew_ref[...] = hn * nout_ref[...]
    hgout_ref[...] = hg


def _last_body(h_ref, mp_ref, nin_ref, nout_ref, wc_ref, bc_ref, wg_ref,
               bg_ref, hgin_ref, w0_ref, b0_ref, w1_ref, b1_ref, w2_ref,
               b2_ref, out_ref):
    _, hg = _layer_core(h_ref, mp_ref, nin_ref, nout_ref, wc_ref, bc_ref,
                        wg_ref, bg_ref, hgin_ref)
    x = jnp.dot(hg, w0_ref[...], preferred_element_type=jnp.float32) + b0_ref[...]
    x = jnp.maximum(x, 0.0)
    x = jnp.dot(x, w1_ref[...], preferred_element_type=jnp.float32) + b1_ref[...]
    x = jnp.maximum(x, 0.0)
    out_ref[...] = jnp.dot(x, w2_ref[...],
                           preferred_element_type=jnp.float32) + b2_ref[...]


_last_call = pl.pallas_call(
    _last_body,
    out_shape=jax.ShapeDtypeStruct((1, D), jnp.float32),
)


_layer_call = pl.pallas_call(
    _layer_body,
    out_shape=[
        jax.ShapeDtypeStruct((N, D), jnp.float32),
        jax.ShapeDtypeStruct((N, D), jnp.float32),
        jax.ShapeDtypeStruct((1, D), jnp.float32),
    ],
)


def _mlp_body(hg_ref, w0_ref, b0_ref, w1_ref, b1_ref, w2_ref, b2_ref,
              out_ref):
    x = hg_ref[...]
    x = jnp.dot(x, w0_ref[...], preferred_element_type=jnp.float32) + b0_ref[...]
    x = jnp.maximum(x, 0.0)
    x = jnp.dot(x, w1_ref[...], preferred_element_type=jnp.float32) + b1_ref[...]
    x = jnp.maximum(x, 0.0)
    out_ref[...] = jnp.dot(x, w2_ref[...],
                           preferred_element_type=jnp.float32) + b2_ref[...]


_mlp_call = pl.pallas_call(
    _mlp_body,
    out_shape=jax.ShapeDtypeStruct((1, D), jnp.float32),
)


def kernel(node_features, edge_index, Wp, bp, Wc0, bc0, Wc1, bc1, Wc2, bc2,
           Wg0, bg0, Wg1, bg1, Wg2, bg2, Wm0, bm0, Wm1, bm1, Wm2, bm2):
    src = edge_index[0].reshape(NW, NCH, CH)
    dst = edge_index[1].reshape(NW, NCH, CH)
    src16 = edge_index[0].reshape(NS, EPT2)
    dst16 = edge_index[1].reshape(NS, EPT2)

    deg4 = _deg_kernel(src, dst).reshape(2 * NC, NH)

    nout_row, nin_row = _norm_call(deg4)
    nout = nout_row.reshape(NH, 1)[:N]
    nin = nin_row.reshape(NH, 1)[:N]

    srcb, dlocb, cnts = _bucket_kernel(src16, dst16)

    h, hs, hg = _proj_call(node_features, Wp, bp.reshape(1, D), nout)

    for Wc, bc, Wg, bg in ((Wc0, bc0, Wg0, bg0),
                           (Wc1, bc1, Wg1, bg1)):
        mp = _seg_kernel(hs, srcb, dlocb, cnts)[:N]
        h, hs, hg = _layer_call(h, mp, nin, nout, Wc, bc.reshape(1, D),
                                Wg, bg.reshape(1, D), hg)

    mp = _seg_kernel(hs, srcb, dlocb, cnts)[:N]
    return _last_call(h, mp, nin, nout, Wc2, bc2.reshape(1, D),
                      Wg2, bg2.reshape(1, D), hg,
                      Wm0, bm0.reshape(1, D), Wm1, bm1.reshape(1, D),
                      Wm2, bm2.reshape(1, D))
